# baseline (speedup 1.0000x reference)
import functools

import numpy as np
import jax
import jax.numpy as jnp
from jax import lax
from jax.experimental import pallas as pl
from jax.experimental.pallas import tpu as pltpu
from jax.experimental.pallas import tpu_sc as plsc

SWAPS = 3
TOTAL = 32768
D = 256

_R1 = np.array([
    [1488030591, 1439099953, 609311445, 260233583, 2118697808, 1156803210,
     1035656343, 1252340714, 2040732033, 1654184288, 625733951, 2086750115,
     1874956968, 2107435338, 909013543, 1372756728],
    [814496280, 34270915, 956997115, 1298601280, 1768113150, 362021218,
     1361115147, 1056098339, 573036096, 962978325, 809066367, 1194074332,
     995758540, 606323265, 1851992991, 1661132541],
    [598165367, 1415523960, 1457916550, 1099422680, 1929759519, 1650016823,
     572115305, 331872980, 355992025, 1585257322, 2054227298, 1414753250,
     442513397, 1800052159, 1325430924, 32135240],
], dtype=np.int32)
_R2 = np.array([
    [1715617077, 264418369, 1417469686, 1457313676, 1352360519, 704757104,
     204966081, 2131313276, 1215959837, 1341945816, 1932178866, 1997354769,
     745677025, 1982421356, 1148378356, 501647516],
    [2011647921, 1141977827, 233273015, 1815371096, 1213686418, 1851131719,
     1053696218, 1906738905, 1205344136, 1973623633, 1332682781, 498722935,
     1227700694, 1792697582, 654972072, 902973260],
    [3148295, 574972484, 1194890849, 831668196, 1051806027, 2105552124,
     619480870, 1217665471, 1968368069, 2036945824, 1286465655, 1900108255,
     1027825450, 1450122370, 1147306558, 449884186],
], dtype=np.int32)

_NC = 2
_NS = 16
_NW = _NC * _NS
_RPW = TOTAL // _NW
_LANES = 16
_NFIX = 2 * SWAPS * _LANES

_GATHER_DNUMS = lax.GatherDimensionNumbers(
    offset_dims=(), collapsed_slice_dims=(0,), start_index_map=(0,))


def _bcast_lane(vec, j):
    idx = jnp.full((_LANES, 1), j, dtype=jnp.int32)
    return lax.gather(vec, idx, _GATHER_DNUMS, (1,),
                      mode=lax.GatherScatterMode.PROMISE_IN_BOUNDS)


def _swap_tables(r1, r2, starts, lens):
    safe = jnp.maximum(lens, 1)
    F = []
    for s in range(SWAPS):
        F.append(starts + r1[s] % safe)
        F.append(starts + r2[s] % safe)
    V = list(F)
    for s in range(SWAPS):
        v1 = V[2 * s]
        v2 = V[2 * s + 1]
        for (g, w) in ((F[2 * s], v2), (F[2 * s + 1], v1)):
            for j in range(_LANES):
                gj = _bcast_lane(g, j)
                wj = _bcast_lane(w, j)
                for t in range(2 * SWAPS):
                    V[t] = jnp.where(F[t] == gj, wj, V[t])
    return F, V


def _sc_body(tbl_hbm, flat_hbm, out_hbm,
             tbl_v, src_v, dst_v, fixrows_v, bsem, fsem, ssem):
    wid = lax.axis_index("s") * _NC + lax.axis_index("c")
    base = wid * _RPW

    bulk = pltpu.async_copy(flat_hbm.at[pl.ds(base, _RPW)],
                            out_hbm.at[pl.ds(base, _RPW)], bsem)

    pltpu.sync_copy(tbl_hbm, tbl_v)
    r1 = [tbl_v[s, :] for s in range(SWAPS)]
    r2 = [tbl_v[SWAPS + s, :] for s in range(SWAPS)]
    starts = tbl_v[2 * SWAPS, :]
    lens = tbl_v[2 * SWAPS + 1, :]

    F, V = _swap_tables(r1, r2, starts, lens)

    bvec = jnp.full((_LANES,), base, dtype=jnp.int32)
    m1 = jnp.full((_LANES,), -1, dtype=jnp.int32)
    for t in range(2 * SWAPS):
        m1 = jnp.where(F[t] == bvec, V[t], m1)
    fillvec = bvec
    for j in range(_LANES):
        cj = _bcast_lane(m1, j)
        fillvec = jnp.where(cj >= 0, cj, fillvec)

    for t in range(2 * SWAPS):
        owned = (F[t] >= base) & (F[t] < base + _RPW)
        src_v[pl.ds(t * _LANES, _LANES)] = jnp.where(owned, V[t], fillvec)
        dst_v[pl.ds(t * _LANES, _LANES)] = jnp.where(owned, F[t], bvec)

    pltpu.async_copy(flat_hbm.at[src_v], fixrows_v, fsem).wait()

    bulk.wait()
    pltpu.async_copy(fixrows_v, out_hbm.at[dst_v], ssem).wait()


_RTBL = np.concatenate([_R1, _R2], axis=0)


@jax.jit
def kernel(flat, cu_seqlens):
    starts = cu_seqlens[:-1]
    lens = cu_seqlens[1:] - starts
    tbl = jnp.concatenate(
        [jnp.asarray(_RTBL), starts[None, :], lens[None, :]], axis=0)
    mesh = plsc.VectorSubcoreMesh(core_axis_name="c", subcore_axis_name="s")
    run = functools.partial(
        pl.kernel,
        mesh=mesh,
        out_type=jax.ShapeDtypeStruct((TOTAL, D), jnp.float32),
        scratch_types=[
            pltpu.VMEM((2 * SWAPS + 2, _LANES), jnp.int32),
            pltpu.VMEM((_NFIX,), jnp.int32),
            pltpu.VMEM((_NFIX,), jnp.int32),
            pltpu.VMEM((_NFIX, D), jnp.float32),
            pltpu.SemaphoreType.DMA,
            pltpu.SemaphoreType.DMA,
            pltpu.SemaphoreType.DMA,
        ],
    )(_sc_body)
    return run(tbl, flat)


# speedup vs baseline: 16.1715x; 16.1715x over previous
import functools

import numpy as np
import jax
import jax.numpy as jnp
from jax import lax
from jax.experimental import pallas as pl
from jax.experimental.pallas import tpu as pltpu
from jax.experimental.pallas import tpu_sc as plsc

SWAPS = 3
TOTAL = 32768
D = 256

_R1 = np.array([
    [1488030591, 1439099953, 609311445, 260233583, 2118697808, 1156803210,
     1035656343, 1252340714, 2040732033, 1654184288, 625733951, 2086750115,
     1874956968, 2107435338, 909013543, 1372756728],
    [814496280, 34270915, 956997115, 1298601280, 1768113150, 362021218,
     1361115147, 1056098339, 573036096, 962978325, 809066367, 1194074332,
     995758540, 606323265, 1851992991, 1661132541],
    [598165367, 1415523960, 1457916550, 1099422680, 1929759519, 1650016823,
     572115305, 331872980, 355992025, 1585257322, 2054227298, 1414753250,
     442513397, 1800052159, 1325430924, 32135240],
], dtype=np.int32)
_R2 = np.array([
    [1715617077, 264418369, 1417469686, 1457313676, 1352360519, 704757104,
     204966081, 2131313276, 1215959837, 1341945816, 1932178866, 1997354769,
     745677025, 1982421356, 1148378356, 501647516],
    [2011647921, 1141977827, 233273015, 1815371096, 1213686418, 1851131719,
     1053696218, 1906738905, 1205344136, 1973623633, 1332682781, 498722935,
     1227700694, 1792697582, 654972072, 902973260],
    [3148295, 574972484, 1194890849, 831668196, 1051806027, 2105552124,
     619480870, 1217665471, 1968368069, 2036945824, 1286465655, 1900108255,
     1027825450, 1450122370, 1147306558, 449884186],
], dtype=np.int32)

_NC = 2
_NS = 16
_NW = _NC * _NS
_RPW = TOTAL // _NW
_LANES = 16
_NFIX = 2 * SWAPS * _LANES
_CHUNK = 128
_NCHUNK = _RPW // _CHUNK

_GATHER_DNUMS = lax.GatherDimensionNumbers(
    offset_dims=(), collapsed_slice_dims=(0,), start_index_map=(0,))


def _bcast_lane(vec, j):
    idx = jnp.full((_LANES, 1), j, dtype=jnp.int32)
    return lax.gather(vec, idx, _GATHER_DNUMS, (1,),
                      mode=lax.GatherScatterMode.PROMISE_IN_BOUNDS)


def _swap_tables(r1, r2, starts, lens):
    safe = jnp.maximum(lens, 1)
    F = []
    for s in range(SWAPS):
        F.append(starts + r1[s] % safe)
        F.append(starts + r2[s] % safe)
    V = list(F)
    for s in range(SWAPS):
        v1 = V[2 * s]
        v2 = V[2 * s + 1]
        for (g, w) in ((F[2 * s], v2), (F[2 * s + 1], v1)):
            for j in range(_LANES):
                gj = _bcast_lane(g, j)
                wj = _bcast_lane(w, j)
                for t in range(2 * SWAPS):
                    V[t] = jnp.where(F[t] == gj, wj, V[t])
    return F, V


def _sc_body(tbl_hbm, flat_hbm, out_hbm,
             tbl_v, src_v, dst_v, fixrows_v, rows_v, gsem, wsem, fsem, ssem):
    wid = lax.axis_index("s") * _NC + lax.axis_index("c")
    base = wid * _RPW

    gd = [None] * _NCHUNK
    wd = [None] * _NCHUNK

    def _rd(c, b):
        return pltpu.async_copy(
            flat_hbm.at[pl.ds(base + c * _CHUNK, _CHUNK)],
            rows_v.at[b], gsem.at[b])

    for c in range(2):
        gd[c] = _rd(c, c & 1)

    pltpu.sync_copy(tbl_hbm, tbl_v)
    r1 = [tbl_v[s, :] for s in range(SWAPS)]
    r2 = [tbl_v[SWAPS + s, :] for s in range(SWAPS)]
    starts = tbl_v[2 * SWAPS, :]
    lens = tbl_v[2 * SWAPS + 1, :]

    F, V = _swap_tables(r1, r2, starts, lens)

    bvec = jnp.full((_LANES,), base, dtype=jnp.int32)
    m1 = jnp.full((_LANES,), -1, dtype=jnp.int32)
    for t in range(2 * SWAPS):
        m1 = jnp.where(F[t] == bvec, V[t], m1)
    fillvec = bvec
    for j in range(_LANES):
        cj = _bcast_lane(m1, j)
        fillvec = jnp.where(cj >= 0, cj, fillvec)

    for t in range(2 * SWAPS):
        owned = (F[t] >= base) & (F[t] < base + _RPW)
        src_v[pl.ds(t * _LANES, _LANES)] = jnp.where(owned, V[t], fillvec)
        dst_v[pl.ds(t * _LANES, _LANES)] = jnp.where(owned, F[t], bvec)

    fix = pltpu.async_copy(flat_hbm.at[src_v], fixrows_v, fsem)

    for c in range(_NCHUNK):
        b = c & 1
        gd[c].wait()
        wd[c] = pltpu.async_copy(rows_v.at[b],
                                 out_hbm.at[pl.ds(base + c * _CHUNK, _CHUNK)],
                                 wsem.at[b])
        if c + 2 < _NCHUNK:
            wd[c].wait()
            gd[c + 2] = _rd(c + 2, b)
    wd[_NCHUNK - 2].wait()
    wd[_NCHUNK - 1].wait()

    fix.wait()
    pltpu.async_copy(fixrows_v, out_hbm.at[dst_v], ssem).wait()


_RTBL = np.concatenate([_R1, _R2], axis=0)


@jax.jit
def kernel(flat, cu_seqlens):
    starts = cu_seqlens[:-1]
    lens = cu_seqlens[1:] - starts
    tbl = jnp.concatenate(
        [jnp.asarray(_RTBL), starts[None, :], lens[None, :]], axis=0)
    mesh = plsc.VectorSubcoreMesh(core_axis_name="c", subcore_axis_name="s")
    run = functools.partial(
        pl.kernel,
        mesh=mesh,
        out_type=jax.ShapeDtypeStruct((TOTAL, D), jnp.float32),
        scratch_types=[
            pltpu.VMEM((2 * SWAPS + 2, _LANES), jnp.int32),
            pltpu.VMEM((_NFIX,), jnp.int32),
            pltpu.VMEM((_NFIX,), jnp.int32),
            pltpu.VMEM((_NFIX, D), jnp.float32),
            pltpu.VMEM((2, _CHUNK, D), jnp.float32),
            pltpu.SemaphoreType.DMA((2,)),
            pltpu.SemaphoreType.DMA((2,)),
            pltpu.SemaphoreType.DMA,
            pltpu.SemaphoreType.DMA,
        ],
    )(_sc_body)
    return run(tbl, flat)
